# baseline (speedup 1.0000x reference)
import jax
import jax.numpy as jnp
from jax import lax
from jax.experimental import pallas as pl
from jax.experimental.pallas import tpu as pltpu
from jax.experimental.pallas import tpu_sc as plsc

_NUM_CORES = 2
_NUM_SUBCORES = 16
_NUM_WORKERS = _NUM_CORES * _NUM_SUBCORES
_CHUNK = 512
_TILES = _CHUNK // 128


def _gather_body(a_hbm, table_hbm, out_hbm,
                 idx0, idx1, rows0, rows1, xb0, xb1,
                 sem_i0, sem_i1, sem_g0, sem_g1, sem_o0, sem_o1):
    idx_v = (idx0, idx1)
    rows_v = (rows0, rows1)
    xbuf = (xb0, xb1)
    sem_i = (sem_i0, sem_i1)
    sem_g = (sem_g0, sem_g1)
    sem_o = (sem_o0, sem_o1)

    c = lax.axis_index("c")
    s = lax.axis_index("s")
    wid = s * _NUM_CORES + c
    batch = a_hbm.shape[1] * a_hbm.shape[3]
    n_chunks = a_hbm.shape[0] * a_hbm.shape[2]
    h_stride = 16 * batch
    tc_stride = 8 * batch
    w_off = wid * (_TILES * 1024)

    lane = lax.iota(jnp.int32, 16)
    cmods = [(lane + d) % 16 for d in range(16)]
    scat_q = [((cm // 8) * (_TILES * 1024) + (cm % 8) * 128 + lane)
              for cm in cmods]

    def idx_load(i, p):
        th = i // 8
        hi = i % 8
        for j in range(_TILES):
            pltpu.async_copy(a_hbm.at[th, wid * _TILES + j, hi, :],
                             idx_v[p].at[pl.ds(j * 128, 128)], sem_i[p])

    def wait_idx(p):
        pltpu.make_async_copy(a_hbm.at[0, 0, 0, :],
                              idx_v[p].at[pl.ds(0, 128)], sem_i[p]).wait()
        pltpu.make_async_copy(a_hbm.at[0, 0, 0, :],
                              idx_v[p].at[pl.ds(0, 128)], sem_i[p]).wait()
        pltpu.make_async_copy(a_hbm.at[0, 0, 0, :],
                              idx_v[p].at[pl.ds(0, 128)], sem_i[p]).wait()
        pltpu.make_async_copy(a_hbm.at[0, 0, 0, :],
                              idx_v[p].at[pl.ds(0, 128)], sem_i[p]).wait()

    def gather_start(p):
        pltpu.async_copy(table_hbm.at[idx_v[p]], rows_v[p], sem_g[p])

    def wait_gather(p):
        pltpu.make_async_copy(table_hbm.at[idx_v[p]],
                              rows_v[p], sem_g[p]).wait()

    def transpose(p):
        @plsc.parallel_loop(0, _CHUNK // 16, step=1, unroll=2)
        def r_body(r0):
            rowidx = lane + r0 * 16
            soff = (r0 // 8) * 1024 + (r0 % 8) * 16
            for d in range(16):
                vec = plsc.load_gather(rows_v[p], [rowidx, cmods[d]])
                plsc.store_scatter(xbuf[p], [scat_q[d] + soff], vec)

    def store_outs(i, p):
        base = i * h_stride + w_off
        pltpu.async_copy(xbuf[p].at[pl.ds(0, _TILES * 1024)],
                         out_hbm.at[pl.ds(base, _TILES * 1024)], sem_o[p])
        pltpu.async_copy(xbuf[p].at[pl.ds(_TILES * 1024, _TILES * 1024)],
                         out_hbm.at[pl.ds(base + tc_stride, _TILES * 1024)],
                         sem_o[p])

    def wait_outs(p):
        pltpu.make_async_copy(out_hbm.at[pl.ds(0, 2 * _TILES * 1024)],
                              xbuf[p], sem_o[p]).wait()

    for p in range(2):
        idx_load(p, p)
    for p in range(2):
        wait_idx(p)
        gather_start(p)

    for p in range(2):
        wait_gather(p)
        idx_load(p + 2, p)
        transpose(p)
        store_outs(p, p)
        wait_idx(p)
        gather_start(p)

    def outer(io, carry):
        i0 = io * 2
        for p in range(2):
            wait_gather(p)
            idx_load(i0 + p + 2, p)
            wait_outs(p)
            transpose(p)
            store_outs(i0 + p, p)
            wait_idx(p)
            gather_start(p)
        return carry

    lax.fori_loop(1, n_chunks // 2 - 1, outer, 0)

    i0 = n_chunks - 2
    for p in range(2):
        wait_gather(p)
        wait_outs(p)
        transpose(p)
        store_outs(i0 + p, p)

    for p in range(2):
        wait_outs(p)


_TA_COLS = 1600


def _ttr_body(tt_hbm, flat_hbm, in0, in1, out0, out1,
              sem_i0, sem_i1, sem_s0, sem_s1):
    ins = (in0, in1)
    outs = (out0, out1)
    sem_i = (sem_i0, sem_i1)
    sem_s = (sem_s0, sem_s1)
    c = lax.axis_index("c")
    s = lax.axis_index("s")
    wid = s * _NUM_CORES + c
    rows = tt_hbm.shape[1]
    n_ch = rows // _TA_COLS

    lane = lax.iota(jnp.int32, 16)
    cmods = [(lane + d) % 16 for d in range(16)]
    sq = [lane * 16 + cm for cm in cmods]

    def load(i, p):
        pltpu.async_copy(tt_hbm.at[:, pl.ds(i * _TA_COLS, _TA_COLS)],
                         ins[p], sem_i[p])

    def wait_load(p):
        pltpu.make_async_copy(tt_hbm.at[:, pl.ds(0, _TA_COLS)],
                              ins[p], sem_i[p]).wait()

    def store(i, p):
        pltpu.async_copy(outs[p],
                         flat_hbm.at[pl.ds(i * _TA_COLS * 16, _TA_COLS * 16)],
                         sem_s[p])

    def wait_store(p):
        pltpu.make_async_copy(flat_hbm.at[pl.ds(0, _TA_COLS * 16)],
                              outs[p], sem_s[p]).wait()

    def transpose(p):
        @plsc.parallel_loop(0, _TA_COLS // 16, step=1, unroll=2)
        def r_body(r0):
            colidx = lane + r0 * 16
            for d in range(16):
                vec = plsc.load_gather(ins[p], [cmods[d], colidx])
                plsc.store_scatter(outs[p], [sq[d] + r0 * 256], vec)

    n_mine = (n_ch - wid + _NUM_WORKERS - 1) // _NUM_WORKERS

    load(wid, 0)

    def chunk_pair(k2, carry):
        i0 = wid + (k2 * 2) * _NUM_WORKERS
        i1 = wid + (k2 * 2 + 1) * _NUM_WORKERS
        wait_load(0)
        load(i1, 1)
        @pl.when(k2 > 0)
        def _():
            wait_store(0)
        transpose(0)
        store(i0, 0)
        wait_load(1)
        @pl.when(i1 + _NUM_WORKERS < n_ch)
        def _():
            load(i1 + _NUM_WORKERS, 0)
        @pl.when(k2 > 0)
        def _():
            wait_store(1)
        transpose(1)
        store(i1, 1)
        return carry

    lax.fori_loop(0, n_mine // 2, chunk_pair, 0)

    @pl.when((n_mine % 2) == 1)
    def _():
        wait_load(0)
        @pl.when(n_mine > 1)
        def _():
            wait_store(0)
        transpose(0)
        store(wid + (n_mine - 1) * _NUM_WORKERS, 0)
        wait_store(0)
        @pl.when(n_mine > 1)
        def _():
            wait_store(1)

    @pl.when((n_mine % 2) == 0)
    def _():
        wait_store(0)
        wait_store(1)


def _transpose_table(table):
    rows, d = table.shape
    tt = jnp.transpose(table)
    mesh = plsc.VectorSubcoreMesh(core_axis_name="c", subcore_axis_name="s")
    flat = pl.kernel(
        _ttr_body,
        out_type=jax.ShapeDtypeStruct((rows * d,), table.dtype),
        mesh=mesh,
        scratch_types=(
            [pltpu.VMEM((16, _TA_COLS), jnp.float32) for _ in range(2)]
            + [pltpu.VMEM((_TA_COLS * 16,), jnp.float32) for _ in range(2)]
            + [pltpu.SemaphoreType.DMA for _ in range(4)]
        ),
        compiler_params=pltpu.CompilerParams(
            use_tc_tiling_on_sc=False, needs_layout_passes=False),
    )(tt)
    return flat.reshape(rows, d)


def kernel(a, table):
    b, h = a.shape
    n = b * h
    d = table.shape[1]
    a4 = a.astype(jnp.int32).reshape(b // 128, 128, h // 8, 8).transpose(2, 0, 3, 1)
    table_rm = _transpose_table(table)
    mesh = plsc.VectorSubcoreMesh(core_axis_name="c", subcore_axis_name="s")
    out = pl.kernel(
        _gather_body,
        out_type=jax.ShapeDtypeStruct((n * d,), table.dtype),
        mesh=mesh,
        scratch_types=(
            [pltpu.VMEM((_CHUNK,), jnp.int32) for _ in range(2)]
            + [pltpu.VMEM((_CHUNK, d), jnp.float32) for _ in range(2)]
            + [pltpu.VMEM((2 * _TILES * 1024,), jnp.float32) for _ in range(2)]
            + [pltpu.SemaphoreType.DMA for _ in range(6)]
        ),
        compiler_params=pltpu.CompilerParams(
            use_tc_tiling_on_sc=False, needs_layout_passes=False),
    )(a4, table_rm)
    x5 = out.reshape(h, 2, b // 128, 8, 128)
    return x5.transpose(2, 4, 0, 1, 3).reshape(b, h, d)


# speedup vs baseline: 2.2608x; 2.2608x over previous
import jax
import jax.numpy as jnp
from jax import lax
from jax.experimental import pallas as pl
from jax.experimental.pallas import tpu as pltpu
from jax.experimental.pallas import tpu_sc as plsc

_NUM_CORES = 2
_NUM_SUBCORES = 16
_NUM_WORKERS = _NUM_CORES * _NUM_SUBCORES
_CHUNK = 512
_TILES = _CHUNK // 128


def _gather_body(a_hbm, table_hbm, out_hbm,
                 idx0, idx1, rows0, rows1, xb0, xb1,
                 sem_i0, sem_i1, sem_g0, sem_g1, sem_o0, sem_o1):
    idx_v = (idx0, idx1)
    rows_v = (rows0, rows1)
    xbuf = (xb0, xb1)
    sem_i = (sem_i0, sem_i1)
    sem_g = (sem_g0, sem_g1)
    sem_o = (sem_o0, sem_o1)

    c = lax.axis_index("c")
    s = lax.axis_index("s")
    wid = s * _NUM_CORES + c
    batch = a_hbm.shape[1] * a_hbm.shape[3]
    n_chunks = a_hbm.shape[0] * a_hbm.shape[2]
    h_stride = 16 * batch
    tc_stride = 8 * batch
    w_off = wid * (_TILES * 1024)

    lane = lax.iota(jnp.int32, 16)
    cmods = [(lane + d) % 16 for d in range(16)]
    scat_q = [((cm // 8) * (_TILES * 1024) + (cm % 8) * 128 + lane)
              for cm in cmods]

    def idx_load(i, p):
        th = i // 8
        hi = i % 8
        for j in range(_TILES):
            pltpu.async_copy(a_hbm.at[th, wid * _TILES + j, hi, :],
                             idx_v[p].at[pl.ds(j * 128, 128)], sem_i[p])

    def wait_idx(p):
        pltpu.make_async_copy(a_hbm.at[0, 0, 0, :],
                              idx_v[p].at[pl.ds(0, 128)], sem_i[p]).wait()
        pltpu.make_async_copy(a_hbm.at[0, 0, 0, :],
                              idx_v[p].at[pl.ds(0, 128)], sem_i[p]).wait()
        pltpu.make_async_copy(a_hbm.at[0, 0, 0, :],
                              idx_v[p].at[pl.ds(0, 128)], sem_i[p]).wait()
        pltpu.make_async_copy(a_hbm.at[0, 0, 0, :],
                              idx_v[p].at[pl.ds(0, 128)], sem_i[p]).wait()

    def gather_start(p):
        pltpu.async_copy(table_hbm.at[idx_v[p]], rows_v[p], sem_g[p])

    def wait_gather(p):
        pltpu.make_async_copy(table_hbm.at[idx_v[p]],
                              rows_v[p], sem_g[p]).wait()

    def transpose(p):
        @plsc.parallel_loop(0, _CHUNK // 16, step=1, unroll=2)
        def r_body(r0):
            rowidx = lane + r0 * 16
            soff = (r0 // 8) * 1024 + (r0 % 8) * 16
            for d in range(16):
                vec = plsc.load_gather(rows_v[p], [rowidx, cmods[d]])
                plsc.store_scatter(xbuf[p], [scat_q[d] + soff], vec)

    def store_outs(i, p):
        base = i * h_stride + w_off
        pltpu.async_copy(xbuf[p].at[pl.ds(0, _TILES * 1024)],
                         out_hbm.at[pl.ds(base, _TILES * 1024)], sem_o[p])
        pltpu.async_copy(xbuf[p].at[pl.ds(_TILES * 1024, _TILES * 1024)],
                         out_hbm.at[pl.ds(base + tc_stride, _TILES * 1024)],
                         sem_o[p])

    def wait_outs(p):
        pltpu.make_async_copy(out_hbm.at[pl.ds(0, 2 * _TILES * 1024)],
                              xbuf[p], sem_o[p]).wait()

    for p in range(2):
        idx_load(p, p)
    for p in range(2):
        wait_idx(p)
        gather_start(p)

    for p in range(2):
        wait_gather(p)
        idx_load(p + 2, p)
        transpose(p)
        store_outs(p, p)
        wait_idx(p)
        gather_start(p)

    def outer(io, carry):
        i0 = io * 2
        for p in range(2):
            wait_gather(p)
            idx_load(i0 + p + 2, p)
            wait_outs(p)
            transpose(p)
            store_outs(i0 + p, p)
            wait_idx(p)
            gather_start(p)
        return carry

    lax.fori_loop(1, n_chunks // 2 - 1, outer, 0)

    i0 = n_chunks - 2
    for p in range(2):
        wait_gather(p)
        wait_outs(p)
        transpose(p)
        store_outs(i0 + p, p)

    for p in range(2):
        wait_outs(p)


def kernel(a, table):
    b, h = a.shape
    n = b * h
    d = table.shape[1]
    a4 = a.astype(jnp.int32).reshape(b // 128, 128, h // 8, 8).transpose(2, 0, 3, 1)
    mesh = plsc.VectorSubcoreMesh(core_axis_name="c", subcore_axis_name="s")
    out = pl.kernel(
        _gather_body,
        out_type=jax.ShapeDtypeStruct((n * d,), table.dtype),
        mesh=mesh,
        scratch_types=(
            [pltpu.VMEM((_CHUNK,), jnp.int32) for _ in range(2)]
            + [pltpu.VMEM((_CHUNK, d), jnp.float32) for _ in range(2)]
            + [pltpu.VMEM((2 * _TILES * 1024,), jnp.float32) for _ in range(2)]
            + [pltpu.SemaphoreType.DMA for _ in range(6)]
        ),
        compiler_params=pltpu.CompilerParams(
            use_tc_tiling_on_sc=False, needs_layout_passes=False),
    )(a4, table)
    x5 = out.reshape(h, 2, b // 128, 8, 128)
    return x5.transpose(2, 4, 0, 1, 3).reshape(b, h, d)


# speedup vs baseline: 2.3910x; 1.0576x over previous
import jax
import jax.numpy as jnp
from jax import lax
from jax.experimental import pallas as pl
from jax.experimental.pallas import tpu as pltpu
from jax.experimental.pallas import tpu_sc as plsc

_NUM_CORES = 2
_NUM_SUBCORES = 16
_NUM_WORKERS = _NUM_CORES * _NUM_SUBCORES
_CHUNK = 1024
_TILES = _CHUNK // 128


def _gather_body(a_hbm, table_hbm, out_hbm,
                 idx0, idx1, rows0, rows1, xb0, xb1,
                 sem_i0, sem_i1, sem_g0, sem_g1, sem_o0, sem_o1):
    idx_v = (idx0, idx1)
    rows_v = (rows0, rows1)
    xbuf = (xb0, xb1)
    sem_i = (sem_i0, sem_i1)
    sem_g = (sem_g0, sem_g1)
    sem_o = (sem_o0, sem_o1)

    c = lax.axis_index("c")
    s = lax.axis_index("s")
    wid = s * _NUM_CORES + c
    batch = a_hbm.shape[1] * a_hbm.shape[3]
    hist = a_hbm.shape[0] * a_hbm.shape[2]
    blocks = batch // _CHUNK
    n_chunks = hist * blocks // _NUM_WORKERS
    u_base = wid * n_chunks
    h_stride = 16 * batch
    tc_stride = 8 * batch

    lane = lax.iota(jnp.int32, 16)
    cmods = [(lane + d) % 16 for d in range(16)]
    scat_q = [((cm // 8) * (_TILES * 1024) + (cm % 8) * 128 + lane)
              for cm in cmods]

    def idx_load(i, p):
        u = u_base + i
        h = u // blocks
        blk = u % blocks
        th = h // 8
        hi = h % 8
        for j in range(_TILES):
            pltpu.async_copy(a_hbm.at[th, blk * _TILES + j, hi, :],
                             idx_v[p].at[pl.ds(j * 128, 128)], sem_i[p])

    def wait_idx(p):
        for _ in range(_TILES):
            pltpu.make_async_copy(a_hbm.at[0, 0, 0, :],
                                  idx_v[p].at[pl.ds(0, 128)], sem_i[p]).wait()

    def gather_start(p):
        pltpu.async_copy(table_hbm.at[idx_v[p]], rows_v[p], sem_g[p])

    def wait_gather(p):
        pltpu.make_async_copy(table_hbm.at[idx_v[p]],
                              rows_v[p], sem_g[p]).wait()

    def transpose(p):
        @plsc.parallel_loop(0, _CHUNK // 16, step=1, unroll=2)
        def r_body(r0):
            rowidx = lane + r0 * 16
            soff = (r0 // 8) * 1024 + (r0 % 8) * 16
            for d in range(16):
                vec = plsc.load_gather(rows_v[p], [rowidx, cmods[d]])
                plsc.store_scatter(xbuf[p], [scat_q[d] + soff], vec)

    def store_outs(i, p):
        u = u_base + i
        base = (u // blocks) * h_stride + (u % blocks) * (_TILES * 1024)
        pltpu.async_copy(xbuf[p].at[pl.ds(0, _TILES * 1024)],
                         out_hbm.at[pl.ds(base, _TILES * 1024)], sem_o[p])
        pltpu.async_copy(xbuf[p].at[pl.ds(_TILES * 1024, _TILES * 1024)],
                         out_hbm.at[pl.ds(base + tc_stride, _TILES * 1024)],
                         sem_o[p])

    def wait_outs(p):
        pltpu.make_async_copy(out_hbm.at[pl.ds(0, 2 * _TILES * 1024)],
                              xbuf[p], sem_o[p]).wait()

    for p in range(2):
        idx_load(p, p)
    for p in range(2):
        wait_idx(p)
        gather_start(p)

    for p in range(2):
        wait_gather(p)
        idx_load(p + 2, p)
        transpose(p)
        store_outs(p, p)
        wait_idx(p)
        gather_start(p)

    def outer(io, carry):
        i0 = io * 2
        for p in range(2):
            wait_gather(p)
            idx_load(i0 + p + 2, p)
            wait_outs(p)
            transpose(p)
            store_outs(i0 + p, p)
            wait_idx(p)
            gather_start(p)
        return carry

    lax.fori_loop(1, n_chunks // 2 - 1, outer, 0)

    i0 = n_chunks - 2
    for p in range(2):
        wait_gather(p)
        wait_outs(p)
        transpose(p)
        store_outs(i0 + p, p)

    for p in range(2):
        wait_outs(p)


def kernel(a, table):
    b, h = a.shape
    n = b * h
    d = table.shape[1]
    a4 = a.astype(jnp.int32).reshape(b // 128, 128, h // 8, 8).transpose(2, 0, 3, 1)
    mesh = plsc.VectorSubcoreMesh(core_axis_name="c", subcore_axis_name="s")
    out = pl.kernel(
        _gather_body,
        out_type=jax.ShapeDtypeStruct((n * d,), table.dtype),
        mesh=mesh,
        scratch_types=(
            [pltpu.VMEM((_CHUNK,), jnp.int32) for _ in range(2)]
            + [pltpu.VMEM((_CHUNK, d), jnp.float32) for _ in range(2)]
            + [pltpu.VMEM((2 * _TILES * 1024,), jnp.float32) for _ in range(2)]
            + [pltpu.SemaphoreType.DMA for _ in range(6)]
        ),
        compiler_params=pltpu.CompilerParams(
            use_tc_tiling_on_sc=False, needs_layout_passes=False),
    )(a4, table)
    x5 = out.reshape(h, 2, b // 128, 8, 128)
    return x5.transpose(2, 4, 0, 1, 3).reshape(b, h, d)
